# Initial kernel scaffold; baseline (speedup 1.0000x reference)
import jax
import jax.numpy as jnp
from jax.experimental import pallas as pl


def kernel(x, edge_index, W, b, gamma, beta):
    raise NotImplementedError("write your pallas kernel here")


# speedup vs baseline: 17.0991x; 17.0991x over previous
import functools

import jax
import jax.numpy as jnp
from jax import lax
from jax.experimental import pallas as pl
from jax.experimental.pallas import tpu as pltpu
from jax.experimental.pallas import tpu_sc as plsc

N_NODES = 10000
N_EDGES = 320000
D = 128
EPS = 1e-5
NEG_SLOPE = 0.01

NC, NS, L = 2, 16, 16
NW = NC * NS
EPW = N_EDGES // NW
CH = 128
NCH = EPW // CH
TAIL = EPW - NCH * CH
NPAD = 10240
RPT = NPAD // NS
NRC = RPT // CH

_mesh = plsc.VectorSubcoreMesh(
    core_axis_name="c", subcore_axis_name="s", num_cores=NC, num_subcores=NS)


@functools.partial(
    pl.kernel,
    out_type=jax.ShapeDtypeStruct((NW, N_NODES), jnp.float32),
    mesh=_mesh,
    compiler_params=pltpu.CompilerParams(needs_layout_passes=False),
    scratch_types=[
        pltpu.VMEM((EPW,), jnp.int32),
        pltpu.VMEM((N_NODES,), jnp.float32),
    ],
)
def _deg_kernel(dst_hbm, deg_hbm, dst_v, deg_v):
    wid = lax.axis_index("s") * NC + lax.axis_index("c")
    pltpu.sync_copy(dst_hbm.at[pl.ds(wid * EPW, EPW)], dst_v)

    zero = jnp.zeros((L,), jnp.float32)

    def _zero(i, c):
        deg_v[pl.ds(i * L, L)] = zero
        return c

    lax.fori_loop(0, N_NODES // L, _zero, 0)

    ones = jnp.ones((L,), jnp.float32)

    def _count(i, c):
        idx = dst_v[pl.ds(i * L, L)]
        plsc.addupdate_scatter(deg_v, [idx], ones)
        return c

    lax.fori_loop(0, EPW // L, _count, 0)
    pltpu.sync_copy(deg_v, deg_hbm.at[wid])


@functools.partial(
    pl.kernel,
    out_type=jax.ShapeDtypeStruct((NC, NPAD, D), jnp.float32),
    mesh=_mesh,
    compiler_params=pltpu.CompilerParams(needs_layout_passes=False),
    scratch_types=[
        pltpu.VMEM((EPW,), jnp.int32),
        pltpu.VMEM((CH,), jnp.int32),
        pltpu.VMEM((CH, D), jnp.float32),
        pltpu.VMEM((TAIL,), jnp.int32),
        pltpu.VMEM((TAIL, D), jnp.float32),
        pltpu.VMEM_SHARED((NPAD, D), jnp.float32),
        pltpu.SemaphoreType.DMA,
    ],
)
def _msg_kernel(src_hbm, dst_hbm, y_hbm, acc_hbm,
                src_v, dsti_v, rows_v, dsti_t, rows_t, acc_s, sem):
    cid = lax.axis_index("c")
    sid = lax.axis_index("s")
    wid = sid * NC + cid
    ebase = wid * EPW

    zero = jnp.zeros((L,), jnp.float32)

    def _zrow(i, c):
        for j in range(D // L):
            rows_v[i, pl.ds(j * L, L)] = zero
        return c

    lax.fori_loop(0, CH, _zrow, 0)
    rbase = sid * RPT
    for k in range(NRC):
        pltpu.sync_copy(rows_v, acc_s.at[pl.ds(rbase + k * CH, CH)])
    plsc.subcore_barrier()

    pltpu.sync_copy(src_hbm.at[pl.ds(ebase, EPW)], src_v)

    def _chunk(i, c):
        cb = i * CH
        pltpu.sync_copy(dst_hbm.at[pl.ds(ebase + cb, CH)], dsti_v)
        pltpu.async_copy(y_hbm.at[src_v.at[pl.ds(cb, CH)]], rows_v, sem).wait()
        pltpu.sync_copy(rows_v, acc_s.at[dsti_v], add=True)
        return c

    lax.fori_loop(0, NCH, _chunk, 0)

    tb = NCH * CH
    pltpu.sync_copy(dst_hbm.at[pl.ds(ebase + tb, TAIL)], dsti_t)
    pltpu.async_copy(y_hbm.at[src_v.at[pl.ds(tb, TAIL)]], rows_t, sem).wait()
    pltpu.sync_copy(rows_t, acc_s.at[dsti_t], add=True)

    plsc.subcore_barrier()

    for k in range(NRC):
        pltpu.sync_copy(acc_s.at[pl.ds(rbase + k * CH, CH)],
                        acc_hbm.at[cid, pl.ds(rbase + k * CH, CH)])


BR = 1000
NB = N_NODES // BR
BR1 = 80
NB1 = N_NODES // BR1


def _lin_body(x_ref, w_ref, degp_ref, y_ref, dis_ref):
    deg = jnp.sum(degp_ref[...], axis=0) + 1.0
    dis = lax.rsqrt(deg)
    xl = jnp.dot(x_ref[...], w_ref[...], preferred_element_type=jnp.float32)
    y_ref[...] = xl * dis
    dis_ref[...] = dis


def _e1_body(a0_ref, a1_ref, y_ref, dis_ref, b_ref, z_ref, s1_ref, s2_ref):
    a = (a0_ref[...] + a1_ref[...]).reshape(BR1, D)
    t = (a + y_ref[...]) * dis_ref[...] + b_ref[...]
    z = jnp.where(t >= 0.0, t, NEG_SLOPE * t)
    z_ref[...] = z
    s1_ref[...] = jnp.sum(z, axis=0).reshape(1, 1, D)
    s2_ref[...] = jnp.sum(z * z, axis=0).reshape(1, 1, D)


def _e2_body(z_ref, s1_ref, s2_ref, g_ref, bt_ref, o_ref):
    n = jnp.float32(N_NODES)
    mean = jnp.sum(s1_ref[...], axis=0) / n
    msq = jnp.sum(s2_ref[...], axis=0) / n
    var = msq - mean * mean
    rstd = lax.rsqrt(var + EPS)
    o_ref[...] = g_ref[...] * (z_ref[...] - mean) * rstd + bt_ref[...]


def kernel(x, edge_index, W, b, gamma, beta):
    src = edge_index[0].astype(jnp.int32)
    dst = edge_index[1].astype(jnp.int32)

    deg_p = _deg_kernel(dst)

    y, dis = pl.pallas_call(
        _lin_body,
        grid=(NB,),
        in_specs=[
            pl.BlockSpec((BR, D), lambda i: (i, 0)),
            pl.BlockSpec((D, D), lambda i: (0, 0)),
            pl.BlockSpec((NW, BR, 1), lambda i: (0, i, 0)),
        ],
        out_specs=[
            pl.BlockSpec((BR, D), lambda i: (i, 0)),
            pl.BlockSpec((BR, 1), lambda i: (i, 0)),
        ],
        out_shape=[
            jax.ShapeDtypeStruct((N_NODES, D), jnp.float32),
            jax.ShapeDtypeStruct((N_NODES, 1), jnp.float32),
        ],
    )(x, W, deg_p.reshape(NW, N_NODES, 1))

    acc = _msg_kernel(src, dst, y)

    z, s1, s2 = pl.pallas_call(
        _e1_body,
        grid=(NB1,),
        in_specs=[
            pl.BlockSpec((1, BR1, D), lambda i: (0, i, 0)),
            pl.BlockSpec((1, BR1, D), lambda i: (1, i, 0)),
            pl.BlockSpec((BR1, D), lambda i: (i, 0)),
            pl.BlockSpec((BR1, 1), lambda i: (i, 0)),
            pl.BlockSpec((1, D), lambda i: (0, 0)),
        ],
        out_specs=[
            pl.BlockSpec((BR1, D), lambda i: (i, 0)),
            pl.BlockSpec((1, 1, D), lambda i: (i, 0, 0)),
            pl.BlockSpec((1, 1, D), lambda i: (i, 0, 0)),
        ],
        out_shape=[
            jax.ShapeDtypeStruct((N_NODES, D), jnp.float32),
            jax.ShapeDtypeStruct((NB1, 1, D), jnp.float32),
            jax.ShapeDtypeStruct((NB1, 1, D), jnp.float32),
        ],
    )(acc, acc, y, dis, b.reshape(1, D))

    out = pl.pallas_call(
        _e2_body,
        grid=(NB,),
        in_specs=[
            pl.BlockSpec((BR, D), lambda i: (i, 0)),
            pl.BlockSpec((NB1, 1, D), lambda i: (0, 0, 0)),
            pl.BlockSpec((NB1, 1, D), lambda i: (0, 0, 0)),
            pl.BlockSpec((1, D), lambda i: (0, 0)),
            pl.BlockSpec((1, D), lambda i: (0, 0)),
        ],
        out_specs=pl.BlockSpec((BR, D), lambda i: (i, 0)),
        out_shape=jax.ShapeDtypeStruct((N_NODES, D), jnp.float32),
    )(z, s1, s2, gamma.reshape(1, D), beta.reshape(1, D))

    return out


# speedup vs baseline: 42.7836x; 2.5021x over previous
import functools

import jax
import jax.numpy as jnp
from jax import lax
from jax.experimental import pallas as pl
from jax.experimental.pallas import tpu as pltpu
from jax.experimental.pallas import tpu_sc as plsc

N_NODES = 10000
N_EDGES = 320000
D = 128
EPS = 1e-5
NEG_SLOPE = 0.01

NC, NS, L = 2, 16, 16
NW = NC * NS
EPW = N_EDGES // NW
CH = 128
NCH = EPW // CH
TAIL = EPW - NCH * CH
NPAD = 10240
RPT = NPAD // NS
NRC = RPT // CH

_mesh = plsc.VectorSubcoreMesh(
    core_axis_name="c", subcore_axis_name="s", num_cores=NC, num_subcores=NS)


@functools.partial(
    pl.kernel,
    out_type=jax.ShapeDtypeStruct((NW, N_NODES), jnp.float32),
    mesh=_mesh,
    compiler_params=pltpu.CompilerParams(needs_layout_passes=False),
    scratch_types=[
        pltpu.VMEM((EPW,), jnp.int32),
        pltpu.VMEM((N_NODES,), jnp.float32),
    ],
)
def _deg_kernel(dst_hbm, deg_hbm, dst_v, deg_v):
    wid = lax.axis_index("s") * NC + lax.axis_index("c")
    pltpu.sync_copy(dst_hbm.at[pl.ds(wid * EPW, EPW)], dst_v)

    zero = jnp.zeros((L,), jnp.float32)

    def _zero(i, c):
        deg_v[pl.ds(i * L, L)] = zero
        return c

    lax.fori_loop(0, N_NODES // L, _zero, 0)

    ones = jnp.ones((L,), jnp.float32)

    def _count(i, c):
        idx = dst_v[pl.ds(i * L, L)]
        plsc.addupdate_scatter(deg_v, [idx], ones)
        return c

    lax.fori_loop(0, EPW // L, _count, 0)
    pltpu.sync_copy(deg_v, deg_hbm.at[wid])


@functools.partial(
    pl.kernel,
    out_type=jax.ShapeDtypeStruct((NC, N_NODES, D), jnp.float32),
    mesh=_mesh,
    compiler_params=pltpu.CompilerParams(needs_layout_passes=False),
    scratch_types=[
        pltpu.VMEM((EPW,), jnp.int32),
        pltpu.VMEM((CH,), jnp.int32),
        pltpu.VMEM((CH,), jnp.int32),
        pltpu.VMEM((CH, D), jnp.float32),
        pltpu.VMEM((CH, D), jnp.float32),
        pltpu.VMEM((TAIL,), jnp.int32),
        pltpu.VMEM((TAIL, D), jnp.float32),
        pltpu.VMEM_SHARED((NPAD, D), jnp.float32),
        pltpu.SemaphoreType.DMA,
        pltpu.SemaphoreType.DMA,
        pltpu.SemaphoreType.DMA,
        pltpu.SemaphoreType.DMA,
    ],
)
def _msg_kernel(src_hbm, dst_hbm, y_hbm, acc_hbm,
                src_v, dsti0, dsti1, rows0, rows1, dsti_t, rows_t, acc_s,
                gsem0, gsem1, dsem0, dsem1):
    cid = lax.axis_index("c")
    sid = lax.axis_index("s")
    wid = sid * NC + cid
    ebase = wid * EPW

    zero = jnp.zeros((L,), jnp.float32)

    def _zrow(i, c):
        for j in range(D // L):
            rows0[i, pl.ds(j * L, L)] = zero
        return c

    lax.fori_loop(0, CH, _zrow, 0)
    rbase = sid * RPT
    for k in range(NRC):
        pltpu.sync_copy(rows0, acc_s.at[pl.ds(rbase + k * CH, CH)])
    plsc.subcore_barrier()

    pltpu.sync_copy(src_hbm.at[pl.ds(ebase, EPW)], src_v)

    def _start(c, dsti, rows, gsem, dsem):
        cb = c * CH
        pltpu.async_copy(dst_hbm.at[pl.ds(ebase + cb, CH)], dsti, dsem)
        pltpu.async_copy(y_hbm.at[src_v.at[pl.ds(cb, CH)]], rows, gsem)

    def _finish(dsti, rows, gsem, dsem):
        pltpu.make_async_copy(dst_hbm.at[pl.ds(ebase, CH)], dsti, dsem).wait()
        pltpu.make_async_copy(y_hbm.at[src_v.at[pl.ds(0, CH)]], rows,
                              gsem).wait()
        pltpu.sync_copy(rows, acc_s.at[dsti], add=True)

    _start(0, dsti0, rows0, gsem0, dsem0)

    def _pair(i2, c):
        ca = 2 * i2
        _start(ca + 1, dsti1, rows1, gsem1, dsem1)
        _finish(dsti0, rows0, gsem0, dsem0)
        _start(ca + 2, dsti0, rows0, gsem0, dsem0)
        _finish(dsti1, rows1, gsem1, dsem1)
        return c

    lax.fori_loop(0, NCH // 2 - 1, _pair, 0)
    _start(NCH - 1, dsti1, rows1, gsem1, dsem1)
    _finish(dsti0, rows0, gsem0, dsem0)
    _finish(dsti1, rows1, gsem1, dsem1)

    tb = NCH * CH
    pltpu.sync_copy(dst_hbm.at[pl.ds(ebase + tb, TAIL)], dsti_t)
    pltpu.async_copy(y_hbm.at[src_v.at[pl.ds(tb, TAIL)]], rows_t, gsem0).wait()
    pltpu.sync_copy(rows_t, acc_s.at[dsti_t], add=True)

    plsc.subcore_barrier()

    ERO = 1000
    if_export = sid < (N_NODES // ERO)
    @pl.when(if_export)
    def _():
        xbase = sid * ERO
        for n0, n in ((0, 128), (128, 128), (256, 128), (384, 128),
                      (512, 128), (640, 128), (768, 128), (896, 104)):
            pltpu.sync_copy(acc_s.at[pl.ds(xbase + n0, n)],
                            acc_hbm.at[cid, pl.ds(xbase + n0, n)])


BR = 1000
NB = N_NODES // BR
DC = 2000


def _dred_body(degp_ref, dsum_ref):
    dsum_ref[...] = jnp.sum(degp_ref[...], axis=0, keepdims=True)


def _lin_body(x_ref, w_ref, deg_ref, y_ref, dis_ref):
    dis = lax.rsqrt(deg_ref[...] + 1.0)
    xl = jnp.dot(x_ref[...], w_ref[...], preferred_element_type=jnp.float32)
    y_ref[...] = xl * dis
    dis_ref[...] = dis


def _e1_body(a0_ref, a1_ref, y_ref, dis_ref, b_ref, z_ref, s1_ref, s2_ref):
    a = (a0_ref[...] + a1_ref[...]).reshape(BR, D)
    t = (a + y_ref[...]) * dis_ref[...] + b_ref[...]
    z = jnp.where(t >= 0.0, t, NEG_SLOPE * t)
    z_ref[...] = z
    s1_ref[...] = jnp.sum(z, axis=0).reshape(1, 1, D)
    s2_ref[...] = jnp.sum(z * z, axis=0).reshape(1, 1, D)


def _e2_body(z_ref, s1_ref, s2_ref, g_ref, bt_ref, o_ref):
    n = jnp.float32(N_NODES)
    mean = jnp.sum(s1_ref[...], axis=0) / n
    msq = jnp.sum(s2_ref[...], axis=0) / n
    var = msq - mean * mean
    rstd = lax.rsqrt(var + EPS)
    o_ref[...] = g_ref[...] * (z_ref[...] - mean) * rstd + bt_ref[...]


def kernel(x, edge_index, W, b, gamma, beta):
    src = edge_index[0].astype(jnp.int32)
    dst = edge_index[1].astype(jnp.int32)

    deg_p = _deg_kernel(dst)

    deg_sum = pl.pallas_call(
        _dred_body,
        grid=(1,),
        in_specs=[pl.BlockSpec((NW, N_NODES), lambda i: (0, 0))],
        out_specs=pl.BlockSpec((1, N_NODES), lambda i: (0, 0)),
        out_shape=jax.ShapeDtypeStruct((1, N_NODES), jnp.float32),
    )(deg_p)

    y, dis = pl.pallas_call(
        _lin_body,
        grid=(NB,),
        in_specs=[
            pl.BlockSpec((BR, D), lambda i: (i, 0)),
            pl.BlockSpec((D, D), lambda i: (0, 0)),
            pl.BlockSpec((BR, 1), lambda i: (i, 0)),
        ],
        out_specs=[
            pl.BlockSpec((BR, D), lambda i: (i, 0)),
            pl.BlockSpec((BR, 1), lambda i: (i, 0)),
        ],
        out_shape=[
            jax.ShapeDtypeStruct((N_NODES, D), jnp.float32),
            jax.ShapeDtypeStruct((N_NODES, 1), jnp.float32),
        ],
    )(x, W, deg_sum.reshape(N_NODES, 1))

    acc = _msg_kernel(src, dst, y)

    z, s1, s2 = pl.pallas_call(
        _e1_body,
        grid=(NB,),
        in_specs=[
            pl.BlockSpec((1, BR, D), lambda i: (0, i, 0)),
            pl.BlockSpec((1, BR, D), lambda i: (1, i, 0)),
            pl.BlockSpec((BR, D), lambda i: (i, 0)),
            pl.BlockSpec((BR, 1), lambda i: (i, 0)),
            pl.BlockSpec((1, D), lambda i: (0, 0)),
        ],
        out_specs=[
            pl.BlockSpec((BR, D), lambda i: (i, 0)),
            pl.BlockSpec((1, 1, D), lambda i: (i, 0, 0)),
            pl.BlockSpec((1, 1, D), lambda i: (i, 0, 0)),
        ],
        out_shape=[
            jax.ShapeDtypeStruct((N_NODES, D), jnp.float32),
            jax.ShapeDtypeStruct((NB, 1, D), jnp.float32),
            jax.ShapeDtypeStruct((NB, 1, D), jnp.float32),
        ],
    )(acc, acc, y, dis, b.reshape(1, D))

    out = pl.pallas_call(
        _e2_body,
        grid=(NB,),
        in_specs=[
            pl.BlockSpec((BR, D), lambda i: (i, 0)),
            pl.BlockSpec((NB, 1, D), lambda i: (0, 0, 0)),
            pl.BlockSpec((NB, 1, D), lambda i: (0, 0, 0)),
            pl.BlockSpec((1, D), lambda i: (0, 0)),
            pl.BlockSpec((1, D), lambda i: (0, 0)),
        ],
        out_specs=pl.BlockSpec((BR, D), lambda i: (i, 0)),
        out_shape=jax.ShapeDtypeStruct((N_NODES, D), jnp.float32),
    )(z, s1, s2, gamma.reshape(1, D), beta.reshape(1, D))

    return out
